# baseline (speedup 1.0000x reference)
import functools
import jax
import jax.numpy as jnp
from jax import lax
from jax.experimental import pallas as pl
from jax.experimental.pallas import tpu as pltpu
from jax.experimental.pallas import tpu_sc as plsc

VOCAB = 100004
EMBED_DIM = 128
BATCH = 4096
HIST_LEN = 200

_TOTAL = BATCH * HIST_LEN
_CHUNK = 128
_NW = 32
_PER_W = _TOTAL // _NW
_NCHUNK = _PER_W // _CHUNK
_NROW = 4


def _gather_body(idx_hbm, table_hbm, out_hbm, idx_v,
                 rows0, rows1, rows2, rows3, shared,
                 sg0, sg1, sg2, sg3, sx0, sx1, sx2, sx3, so0, so1):
    wid = lax.axis_index("s") * 2 + lax.axis_index("c")
    sid = lax.axis_index("s")
    base = wid * _PER_W

    pltpu.sync_copy(idx_hbm.at[pl.ds(base, _PER_W)], idx_v)

    rows = (rows0, rows1, rows2, rows3)
    sg = (sg0, sg1, sg2, sg3)
    sx = (sx0, sx1, sx2, sx3)
    so = (so0, so1)

    def gather_start(g, b):
        pltpu.async_copy(
            table_hbm.at[idx_v.at[pl.ds(g * _CHUNK, _CHUNK)]], rows[b], sg[b]
        )

    def wait_gather(b):
        pltpu.make_async_copy(
            table_hbm.at[idx_v.at[pl.ds(0, _CHUNK)]], rows[b], sg[b]
        ).wait()

    def xbar_start(b, s):
        pltpu.async_copy(rows[b], shared_at(s), sx[b])

    def wait_xbar(b, s):
        pltpu.make_async_copy(rows[b], shared_at(s), sx[b]).wait()

    def out_start(g, s):
        pltpu.async_copy(
            shared_at(s), out_hbm.at[pl.ds(base + g * _CHUNK, _CHUNK)], so[s]
        )

    def wait_out(s):
        pltpu.make_async_copy(
            shared_at(s), out_hbm.at[pl.ds(base, _CHUNK)], so[s]
        ).wait()

    def shared_at(s):
        return shared.at[sid, s]

    gather_start(0, 0)
    gather_start(1, 1)
    gather_start(2, 2)
    wait_gather(0)
    xbar_start(0, 0)
    gather_start(3, 3)
    wait_gather(1)
    xbar_start(1, 1)
    wait_xbar(0, 0)
    out_start(0, 0)
    gather_start(4, 0)

    @pl.loop(2, _NCHUNK - 6, step=_NROW)
    def _(g0):
        for k in range(_NROW):
            g = g0 + k
            b = (2 + k) % _NROW
            s = k % 2
            ob = (1 + k) % _NROW
            os_ = (1 + k) % 2
            wait_gather(b)
            wait_out(s)
            xbar_start(b, s)
            wait_xbar(ob, os_)
            out_start(g - 1, os_)
            gather_start(g + 3, (k + 1) % _NROW)

    for g in (_NCHUNK - 6, _NCHUNK - 5, _NCHUNK - 4):
        b = g % _NROW
        s = g % 2
        wait_gather(b)
        wait_out(s)
        xbar_start(b, s)
        wait_xbar((g - 1) % _NROW, (g - 1) % 2)
        out_start(g - 1, (g - 1) % 2)
        gather_start(g + 3, (g + 3) % _NROW)

    for g in (_NCHUNK - 3, _NCHUNK - 2, _NCHUNK - 1):
        b = g % _NROW
        s = g % 2
        wait_gather(b)
        wait_out(s)
        xbar_start(b, s)
        wait_xbar((g - 1) % _NROW, (g - 1) % 2)
        out_start(g - 1, (g - 1) % 2)

    g_last = _NCHUNK - 1
    wait_xbar(g_last % _NROW, g_last % 2)
    out_start(g_last, g_last % 2)
    wait_out(0)
    wait_out(1)


def kernel(input_indices, embedding_matrix):
    idx_flat = input_indices.reshape(_TOTAL)

    mesh = plsc.VectorSubcoreMesh(core_axis_name="c", subcore_axis_name="s")

    out_flat = pl.kernel(
        _gather_body,
        mesh=mesh,
        out_type=jax.ShapeDtypeStruct((_TOTAL, EMBED_DIM), jnp.float32),
        scratch_types=(
            [pltpu.VMEM((_PER_W,), jnp.int32)]
            + [pltpu.VMEM((_CHUNK, EMBED_DIM), jnp.float32)] * _NROW
            + [pltpu.VMEM_SHARED((16, 2, _CHUNK, EMBED_DIM), jnp.float32)]
            + [pltpu.SemaphoreType.DMA] * 10
        ),
    )(idx_flat, embedding_matrix)

    return out_flat.reshape(BATCH, HIST_LEN, EMBED_DIM)
